# baseline (speedup 1.0000x reference)
import functools

import jax
import jax.numpy as jnp
from jax import lax
from jax.experimental import pallas as pl
from jax.experimental.pallas import tpu as pltpu
from jax.experimental.pallas import tpu_sc as plsc

_VOCAB = 1000000
_DIM = 64
_BATCH = 16384

_INFO = plsc.get_sparse_core_info()
_NC = _INFO.num_cores
_NS = _INFO.num_subcores
_NW = _NC * _NS
_L = _INFO.num_lanes
_N_PER_W = _BATCH // _NW
_IDX_CHUNK = 128
_N_CHUNKS = _N_PER_W // _IDX_CHUNK
_GROUPS = _N_PER_W // _L


def _rsqrt_newton(x):
    i = lax.bitcast_convert_type(x, jnp.int32)
    i = jnp.int32(0x5F3759DF) - lax.shift_right_logical(i, 1)
    y = lax.bitcast_convert_type(i, jnp.float32)
    xh = x * jnp.float32(0.5)
    for _ in range(3):
        y = y * (jnp.float32(1.5) - xh * y * y)
    return y


def _normalize_rows(rows_ref):
    iota = lax.broadcasted_iota(jnp.int32, (_L,), 0)

    def group_body(g, carry):
        row = g * _L + iota

        def ss_body(c, acc):
            cs = jnp.full((_L,), c, dtype=jnp.int32)
            x = plsc.load_gather(rows_ref, [row, cs])
            return acc + x * x

        ss = lax.fori_loop(0, _DIM, ss_body, jnp.zeros((_L,), jnp.float32),
                           unroll=8)
        inv = _rsqrt_newton(ss)

        def scale_body(c, carry2):
            cs = jnp.full((_L,), c, dtype=jnp.int32)
            x = plsc.load_gather(rows_ref, [row, cs])
            plsc.store_scatter(rows_ref, [row, cs], x * inv)
            return carry2

        lax.fori_loop(0, _DIM, scale_body, 0, unroll=8)
        return carry

    lax.fori_loop(0, _GROUPS, group_body, 0)


@functools.partial(
    pl.kernel,
    out_type=jax.ShapeDtypeStruct((2, _BATCH, _DIM), jnp.float32),
    mesh=plsc.VectorSubcoreMesh(core_axis_name="c", subcore_axis_name="s"),
    compiler_params=pltpu.CompilerParams(
        needs_layout_passes=False, use_tc_tiling_on_sc=False
    ),
    scratch_types=[
        pltpu.VMEM((_N_CHUNKS, _IDX_CHUNK), jnp.int32),
        pltpu.VMEM((_N_CHUNKS, _IDX_CHUNK), jnp.int32),
        pltpu.VMEM((_N_PER_W, _DIM), jnp.float32),
        pltpu.VMEM((_N_PER_W, _DIM), jnp.float32),
        pltpu.SemaphoreType.DMA,
        pltpu.SemaphoreType.DMA,
    ],
)
def _sc_kernel(in_data, out_data, in_table, out_table, out,
               idx0, idx1, rows0, rows1, sem0, sem1):
    wid = lax.axis_index("s") * _NC + lax.axis_index("c")
    base = wid * _N_PER_W

    crow = wid * _N_CHUNKS
    pltpu.sync_copy(in_data.at[pl.ds(crow, _N_CHUNKS)], idx0)
    pltpu.sync_copy(out_data.at[pl.ds(crow, _N_CHUNKS)], idx1)

    g0 = [
        pltpu.async_copy(in_table.at[idx0.at[j]],
                         rows0.at[pl.ds(j * _IDX_CHUNK, _IDX_CHUNK)], sem0)
        for j in range(_N_CHUNKS)
    ]
    g1 = [
        pltpu.async_copy(out_table.at[idx1.at[j]],
                         rows1.at[pl.ds(j * _IDX_CHUNK, _IDX_CHUNK)], sem1)
        for j in range(_N_CHUNKS)
    ]

    for g in g0:
        g.wait()
    _normalize_rows(rows0)
    o0 = pltpu.async_copy(rows0, out.at[0, pl.ds(base, _N_PER_W)], sem0)

    for g in g1:
        g.wait()
    _normalize_rows(rows1)
    o1 = pltpu.async_copy(rows1, out.at[1, pl.ds(base, _N_PER_W)], sem1)

    o0.wait()
    o1.wait()


def kernel(in_data, out_data, in_table, out_table):
    in2 = in_data.astype(jnp.int32).reshape(_BATCH // _IDX_CHUNK, _IDX_CHUNK)
    out2 = out_data.astype(jnp.int32).reshape(_BATCH // _IDX_CHUNK, _IDX_CHUNK)
    return _sc_kernel(in2, out2, in_table, out_table)


# speedup vs baseline: 1.4916x; 1.4916x over previous
import functools

import jax
import jax.numpy as jnp
from jax import lax
from jax.experimental import pallas as pl
from jax.experimental.pallas import tpu as pltpu
from jax.experimental.pallas import tpu_sc as plsc

_VOCAB = 1000000
_DIM = 64
_BATCH = 16384

_INFO = plsc.get_sparse_core_info()
_NC = _INFO.num_cores
_NS = _INFO.num_subcores
_NW = _NC * _NS
_L = _INFO.num_lanes
_N_PER_W = _BATCH // _NW
_GROUPS = _N_PER_W // _L
_PACK = 128 // _DIM
_PROWS = _N_PER_W // _PACK


def _rsqrt_newton(x):
    i = lax.bitcast_convert_type(x, jnp.int32)
    i = jnp.int32(0x5F3759DF) - lax.shift_right_logical(i, 1)
    y = lax.bitcast_convert_type(i, jnp.float32)
    xh = x * jnp.float32(0.5)
    for _ in range(3):
        y = y * (jnp.float32(1.5) - xh * y * y)
    return y


def _normalize_rows(rows_ref):
    iota = lax.broadcasted_iota(jnp.int32, (_L,), 0)

    def group_body(g, carry):
        rr = g * _L + iota
        prow = lax.shift_right_logical(rr, 1)
        pcol = lax.shift_left(rr & 1, 6)

        def ss_body(c, acc):
            x = plsc.load_gather(rows_ref, [prow, pcol + c])
            return acc + x * x

        ss = lax.fori_loop(0, _DIM, ss_body, jnp.zeros((_L,), jnp.float32),
                           unroll=8)
        inv = _rsqrt_newton(ss)

        def scale_body(c, carry2):
            x = plsc.load_gather(rows_ref, [prow, pcol + c])
            plsc.store_scatter(rows_ref, [prow, pcol + c], x * inv)
            return carry2

        lax.fori_loop(0, _DIM, scale_body, 0, unroll=8)
        return carry

    lax.fori_loop(0, _GROUPS, group_body, 0)


def _issue_row_gathers(table, idx_ref, rows_ref, sem):
    def g_body(g, carry):
        iv = idx_ref[pl.ds(g * _L, _L)]
        for j in range(_L):
            r = g * _L + j
            pltpu.async_copy(
                table.at[iv[j]],
                rows_ref.at[r >> 1, pl.ds((r & 1) * _DIM, _DIM)],
                sem,
            )
        return carry

    lax.fori_loop(0, _GROUPS, g_body, 0)


@functools.partial(
    pl.kernel,
    out_type=jax.ShapeDtypeStruct((2, _BATCH // _PACK, 128), jnp.float32),
    mesh=plsc.VectorSubcoreMesh(core_axis_name="c", subcore_axis_name="s"),
    compiler_params=pltpu.CompilerParams(needs_layout_passes=False),
    scratch_types=[
        pltpu.VMEM((_N_PER_W,), jnp.int32),
        pltpu.VMEM((_N_PER_W,), jnp.int32),
        pltpu.VMEM((_PROWS, 128), jnp.float32),
        pltpu.VMEM((_PROWS, 128), jnp.float32),
        pltpu.SemaphoreType.DMA,
        pltpu.SemaphoreType.DMA,
        pltpu.SemaphoreType.DMA,
    ],
)
def _sc_kernel(in_data, out_data, in_table, out_table, out,
               idx0, idx1, rows0, rows1, sem0, sem1, osem):
    wid = lax.axis_index("s") * _NC + lax.axis_index("c")
    base = wid * _N_PER_W
    pbase = wid * _PROWS

    pltpu.sync_copy(in_data.at[pl.ds(base, _N_PER_W)], idx0)
    pltpu.sync_copy(out_data.at[pl.ds(base, _N_PER_W)], idx1)

    _issue_row_gathers(in_table, idx0, rows0, sem0)
    _issue_row_gathers(out_table, idx1, rows1, sem1)

    pltpu.make_async_copy(out.at[0, pl.ds(0, _PROWS)], rows0, sem0).wait()
    _normalize_rows(rows0)
    o0 = pltpu.async_copy(rows0, out.at[0, pl.ds(pbase, _PROWS)], osem)

    pltpu.make_async_copy(out.at[1, pl.ds(0, _PROWS)], rows1, sem1).wait()
    _normalize_rows(rows1)
    o1 = pltpu.async_copy(rows1, out.at[1, pl.ds(pbase, _PROWS)], osem)

    o0.wait()
    o1.wait()


def kernel(in_data, out_data, in_table, out_table):
    packed = _sc_kernel(
        in_data.astype(jnp.int32), out_data.astype(jnp.int32),
        in_table, out_table,
    )
    return packed.reshape(2, _BATCH, _DIM)
